# baseline (speedup 1.0000x reference)
import functools

import jax
import jax.numpy as jnp
from jax.experimental import pallas as pl
from jax.experimental.pallas import tpu as pltpu

Q = 900
C = 91
K = 300
KPAD = 384
QPAD = 904
NT = QPAD // 8
BIG = 0x7FFFFFFF


def _topk_body(logits_ref, pmt_ref, scores_ref, labels_ref, bidx_ref,
               s_ref, tmax_ref, tidx_ref, sbits_ref, fidx_ref):
    x = logits_ref[0]
    sig = jax.nn.sigmoid(x)
    prob = jnp.dot(sig, pmt_ref[...], preferred_element_type=jnp.float32)
    col = jax.lax.broadcasted_iota(jnp.int32, (Q, 128), 1)
    bits = jnp.where(col < C,
                     jax.lax.bitcast_convert_type(prob, jnp.int32),
                     jnp.int32(-1))
    s_ref[...] = jnp.full((QPAD, 128), -1, jnp.int32)
    s_ref[0:Q, :] = bits

    lane1 = jax.lax.broadcasted_iota(jnp.int32, (1, 128), 1)
    rr = jax.lax.broadcasted_iota(jnp.int32, (8, 128), 0)
    cc = jax.lax.broadcasted_iota(jnp.int32, (8, 128), 1)

    tmax_ref[...] = jnp.full((1, 128), -1, jnp.int32)
    tidx_ref[...] = jnp.full((1, 128), BIG, jnp.int32)

    def init_tile(t, _):
        tile = s_ref[pl.ds(t * 8, 8), :]
        m = jnp.max(tile)
        fidx = (t * 8 + rr) * C + cc
        mi = jnp.min(jnp.where(tile == m, fidx, BIG))
        tmax_ref[...] = jnp.where(lane1 == t, m, tmax_ref[...])
        tidx_ref[...] = jnp.where(lane1 == t, mi, tidx_ref[...])
        return 0

    jax.lax.fori_loop(0, NT, init_tile, 0)

    oi = (jax.lax.broadcasted_iota(jnp.int32, (3, 128), 0) * 128
          + jax.lax.broadcasted_iota(jnp.int32, (3, 128), 1))
    sbits_ref[...] = jnp.zeros((3, 128), jnp.int32)
    fidx_ref[...] = jnp.zeros((3, 128), jnp.int32)

    def step(j, _):
        tm = tmax_ref[...]
        m = jnp.max(tm)
        i = jnp.min(jnp.where(tm == m, tidx_ref[...], BIG))
        sel = oi == j
        sbits_ref[...] = jnp.where(sel, m, sbits_ref[...])
        fidx_ref[...] = jnp.where(sel, i, fidx_ref[...])
        q = i // C
        c = i - q * C
        t = q // 8
        r = q - t * 8
        tile = s_ref[pl.ds(t * 8, 8), :]
        tile = jnp.where((rr == r) & (cc == c), jnp.int32(-1), tile)
        s_ref[pl.ds(t * 8, 8), :] = tile
        mt = jnp.max(tile)
        fidx = (t * 8 + rr) * C + cc
        mi = jnp.min(jnp.where(tile == mt, fidx, BIG))
        tmax_ref[...] = jnp.where(lane1 == t, mt, tmax_ref[...])
        tidx_ref[...] = jnp.where(lane1 == t, mi, tidx_ref[...])
        return 0

    jax.lax.fori_loop(0, K, step, 0)

    idx = fidx_ref[...]
    scores_ref[0] = jax.lax.bitcast_convert_type(sbits_ref[...], jnp.float32)
    labels_ref[0] = idx % C
    bidx_ref[0] = idx // C


@functools.partial(jax.jit, static_argnames=())
def _topk_call(pred_logits, pmt):
    out_shapes = (
        jax.ShapeDtypeStruct((16, 3, 128), jnp.float32),
        jax.ShapeDtypeStruct((16, 3, 128), jnp.int32),
        jax.ShapeDtypeStruct((16, 3, 128), jnp.int32),
    )
    return pl.pallas_call(
        _topk_body,
        grid=(16,),
        in_specs=[
            pl.BlockSpec((1, Q, 256), lambda b: (b, 0, 0)),
            pl.BlockSpec((256, 128), lambda b: (0, 0)),
        ],
        out_specs=(
            pl.BlockSpec((1, 3, 128), lambda b: (b, 0, 0)),
            pl.BlockSpec((1, 3, 128), lambda b: (b, 0, 0)),
            pl.BlockSpec((1, 3, 128), lambda b: (b, 0, 0)),
        ),
        out_shape=out_shapes,
        scratch_shapes=[
            pltpu.VMEM((QPAD, 128), jnp.int32),
            pltpu.VMEM((1, 128), jnp.int32),
            pltpu.VMEM((1, 128), jnp.int32),
            pltpu.VMEM((3, 128), jnp.int32),
            pltpu.VMEM((3, 128), jnp.int32),
        ],
        compiler_params=pltpu.CompilerParams(
            dimension_semantics=("arbitrary",),
        ),
    )(pred_logits, pmt)


def kernel(pred_logits, pred_boxes, target_sizes, positive_map):
    pmt = jnp.zeros((256, 128), jnp.float32).at[:, :C].set(positive_map.T)
    scores3, labels3, bidx3 = _topk_call(pred_logits, pmt)
    scores = scores3.reshape(16, KPAD)[:, :K]
    labels = labels3.reshape(16, KPAD)[:, :K]
    bidx = bidx3.reshape(16, KPAD)[:, :K]

    b = pred_boxes
    cx, cy, w, h = b[..., 0], b[..., 1], b[..., 2], b[..., 3]
    boxes = jnp.stack([cx - 0.5 * w, cy - 0.5 * h,
                       cx + 0.5 * w, cy + 0.5 * h], axis=-1)
    idx = jnp.repeat(bidx[:, :, None], 4, axis=-1)
    boxes = jnp.take_along_axis(boxes, idx, axis=1)
    img_h = target_sizes[:, 0].astype(boxes.dtype)
    img_w = target_sizes[:, 1].astype(boxes.dtype)
    scale_fct = jnp.stack([img_w, img_h, img_w, img_h], axis=1)
    boxes = boxes * scale_fct[:, None, :]
    return scores, labels, boxes


# speedup vs baseline: 3.4564x; 3.4564x over previous
import functools

import jax
import jax.numpy as jnp
from jax import lax
from jax.experimental import pallas as pl
from jax.experimental.pallas import tpu as pltpu
from jax.experimental.pallas import tpu_sc as plsc

Q = 900
C = 91
K = 300
KPAD = 384
QPAD = 1024
B = 16
BIGF = 1e9
NEG = -2.0


def _prep_body(logits_ref, pmt_ref, s_out, rmax_out, ridx_out):
    x = logits_ref[0]
    sig = jax.nn.sigmoid(x)
    prob = jnp.dot(sig, pmt_ref[...], preferred_element_type=jnp.float32)
    col = jax.lax.broadcasted_iota(jnp.int32, (Q, 128), 1)
    vals = jnp.where(col < C, prob, jnp.float32(-1.0))
    s_out[0] = jnp.full((QPAD, 128), -1.0, jnp.float32)
    s_out[0, 0:Q, :] = vals

    d3 = s_out[0].reshape(8, 128, 128)
    s3 = jax.lax.broadcasted_iota(jnp.int32, (8, 128, 128), 0)
    m3 = jax.lax.broadcasted_iota(jnp.int32, (8, 128, 128), 1)
    l3 = jax.lax.broadcasted_iota(jnp.int32, (8, 128, 128), 2)
    fidx3 = ((s3 * 128 + m3) * C + l3).astype(jnp.float32)
    rmax = jnp.max(d3, axis=2)
    cand = jnp.where(d3 == rmax[:, :, None], fidx3, jnp.float32(BIGF))
    ridx = jnp.min(cand, axis=2)
    rmax_out[0] = rmax
    ridx_out[0] = ridx


def _prep_call(pred_logits, pmt):
    out_shapes = (
        jax.ShapeDtypeStruct((B, QPAD, 128), jnp.float32),
        jax.ShapeDtypeStruct((B, 8, 128), jnp.float32),
        jax.ShapeDtypeStruct((B, 8, 128), jnp.float32),
    )
    return pl.pallas_call(
        _prep_body,
        grid=(B,),
        in_specs=[
            pl.BlockSpec((1, Q, 256), lambda b: (b, 0, 0)),
            pl.BlockSpec((256, 128), lambda b: (0, 0)),
        ],
        out_specs=(
            pl.BlockSpec((1, QPAD, 128), lambda b: (b, 0, 0)),
            pl.BlockSpec((1, 8, 128), lambda b: (b, 0, 0)),
            pl.BlockSpec((1, 8, 128), lambda b: (b, 0, 0)),
        ),
        out_shape=out_shapes,
        compiler_params=pltpu.CompilerParams(
            dimension_semantics=("arbitrary",),
        ),
    )(pred_logits, pmt)


def _topk_body(s_ref, rmax_in, ridx_in, scores_ref, labels_ref, bidx_ref,
               smax_ref, sidx_ref):
    lane1 = jax.lax.broadcasted_iota(jnp.int32, (1, 128), 1)
    col128 = jax.lax.broadcasted_iota(jnp.int32, (B, 128), 1)

    smax_ref[...] = rmax_in[...]
    sidx_ref[...] = ridx_in[...]

    def make_step(chunk):
        lo = chunk * 128

        def step(j, _):
            S = smax_ref[...]
            I = sidx_ref[...]
            m16 = jnp.max(S, axis=(1, 2), keepdims=True)
            cand = jnp.where(S == m16, I, jnp.float32(BIGF))
            i16 = jnp.min(cand, axis=(1, 2), keepdims=True)
            sel = col128 == (j - lo)
            scores_ref[:, lo:lo + 128] = jnp.where(
                sel, m16.reshape(B, 1), scores_ref[:, lo:lo + 128])
            labels_ref[:, lo:lo + 128] = jnp.where(
                sel, i16.reshape(B, 1), labels_ref[:, lo:lo + 128])
            new_s = []
            new_i = []
            for b in range(B):
                ib = i16[b]
                mb = m16[b]
                r = ib[0, 0].astype(jnp.int32) // C
                row = s_ref[b, pl.ds(r, 1), :]
                lidx = (r * C + lane1).astype(jnp.float32)
                rem = (row < mb) | ((row == mb) & (lidx > ib))
                masked = jnp.where(rem, row, jnp.float32(NEG))
                nm = jnp.max(masked, keepdims=True)
                ncand = jnp.where(masked == nm, lidx, jnp.float32(BIGF))
                ni = jnp.min(ncand, keepdims=True)
                win = (S[b] == mb) & (I[b] == ib)
                new_s.append(jnp.where(win, nm, S[b])[None])
                new_i.append(jnp.where(win, ni, I[b])[None])
            smax_ref[...] = jnp.concatenate(new_s, axis=0)
            sidx_ref[...] = jnp.concatenate(new_i, axis=0)
            return 0

        return step

    scores_ref[...] = jnp.zeros((B, KPAD), jnp.float32)
    labels_ref[...] = jnp.zeros((B, KPAD), jnp.float32)
    jax.lax.fori_loop(0, 128, make_step(0), 0)
    jax.lax.fori_loop(128, 256, make_step(1), 0)
    jax.lax.fori_loop(256, K, make_step(2), 0)

    idx = labels_ref[...].astype(jnp.int32)
    labels_ref[...] = (idx % C).astype(jnp.float32)
    bidx_ref[...] = idx // C


def _topk_call(s_all, rmax_all, ridx_all):
    out_shapes = (
        jax.ShapeDtypeStruct((B, KPAD), jnp.float32),
        jax.ShapeDtypeStruct((B, KPAD), jnp.float32),
        jax.ShapeDtypeStruct((B, KPAD), jnp.int32),
    )
    return pl.pallas_call(
        _topk_body,
        out_shape=out_shapes,
        scratch_shapes=[
            pltpu.VMEM((B, 8, 128), jnp.float32),
            pltpu.VMEM((B, 8, 128), jnp.float32),
        ],
    )(s_all, rmax_all, ridx_all)


HALF = KPAD // 2


def _box_body(boxes_hbm, bidx_hbm, ts_hbm, out_hbm, boxes_v, idx_v, ts_v, out_v):
    b = lax.axis_index("s")
    half = lax.axis_index("c")
    pltpu.sync_copy(boxes_hbm.at[pl.ds(b * Q * 4, Q * 4)], boxes_v)
    pltpu.sync_copy(bidx_hbm.at[pl.ds(b * KPAD + half * HALF, HALF)], idx_v)
    pltpu.sync_copy(ts_hbm, ts_v)
    hsz = plsc.load_gather(ts_v, [jnp.full((16,), 2 * b, jnp.int32)]
                           ).astype(jnp.float32)
    wsz = plsc.load_gather(ts_v, [jnp.full((16,), 2 * b + 1, jnp.int32)]
                           ).astype(jnp.float32)
    lane = lax.broadcasted_iota(jnp.int32, (16,), 0)
    for j in range(HALF // 16):
        qi = idx_v[pl.ds(j * 16, 16)] * 4
        cx = plsc.load_gather(boxes_v, [qi])
        cy = plsc.load_gather(boxes_v, [qi + 1])
        w = plsc.load_gather(boxes_v, [qi + 2])
        h = plsc.load_gather(boxes_v, [qi + 3])
        rows = (lane + j * 16) * 4
        plsc.store_scatter(out_v, [rows], (cx - 0.5 * w) * wsz)
        plsc.store_scatter(out_v, [rows + 1], (cy - 0.5 * h) * hsz)
        plsc.store_scatter(out_v, [rows + 2], (cx + 0.5 * w) * wsz)
        plsc.store_scatter(out_v, [rows + 3], (cy + 0.5 * h) * hsz)
    pltpu.sync_copy(out_v,
                    out_hbm.at[pl.ds((b * KPAD + half * HALF) * 4, HALF * 4)])


def _box_call(pred_boxes, bidx_pad, target_sizes):
    mesh = plsc.VectorSubcoreMesh(core_axis_name="c", subcore_axis_name="s")
    out = pl.kernel(
        _box_body,
        mesh=mesh,
        out_type=jax.ShapeDtypeStruct((B * KPAD * 4,), jnp.float32),
        scratch_types=[
            pltpu.VMEM((Q * 4,), jnp.float32),
            pltpu.VMEM((HALF,), jnp.int32),
            pltpu.VMEM((32,), jnp.int32),
            pltpu.VMEM((HALF * 4,), jnp.float32),
        ],
        compiler_params=pltpu.CompilerParams(needs_layout_passes=False),
    )(pred_boxes.reshape(-1), bidx_pad.reshape(-1), target_sizes.reshape(-1))
    return out.reshape(B, KPAD, 4)


def kernel(pred_logits, pred_boxes, target_sizes, positive_map):
    pmt = jnp.zeros((256, 128), jnp.float32).at[:, :C].set(positive_map.T)
    s_all, rmax_all, ridx_all = _prep_call(pred_logits, pmt)
    scores, labels, bidx = _topk_call(s_all, rmax_all, ridx_all)
    boxes_pad = _box_call(pred_boxes, bidx, target_sizes)
    return (scores[:, :K], labels[:, :K].astype(jnp.int32),
            boxes_pad[:, :K, :])
